# Initial kernel scaffold; baseline (speedup 1.0000x reference)
import jax
import jax.numpy as jnp
from jax.experimental import pallas as pl


def kernel(xs, table):
    raise NotImplementedError("write your pallas kernel here")


# speedup vs baseline: 2.5381x; 2.5381x over previous
import functools

import jax
import jax.numpy as jnp
from jax import lax
from jax.experimental import pallas as pl
from jax.experimental.pallas import tpu as pltpu
from jax.experimental.pallas import tpu_sc as plsc

BATCH = 16384
HIST = 50
DIM = 64
LANES = 16
NUM_WORKERS = 32
SAMPLES_PER_TILE = BATCH // NUM_WORKERS
CHUNK = 2
IDX_PER_CHUNK = CHUNK * HIST
NCHUNK = SAMPLES_PER_TILE // CHUNK
INV_HIST = 1.0 / HIST


def _sc_body(xs_hbm, table_hbm, out_hbm, idx_v, rows0, rows1, out_v, sem0, sem1):
    cid = lax.axis_index("c")
    sid = lax.axis_index("s")
    wid = sid * 2 + cid

    pltpu.sync_copy(xs_hbm.at[wid], idx_v)

    def start_gather(j, buf, sem):
        pltpu.async_copy(table_hbm.at[idx_v.at[j]], buf, sem)

    def wait_gather(j, buf, sem):
        pltpu.make_async_copy(table_hbm.at[idx_v.at[j]], buf, sem).wait()

    def reduce_chunk(j, buf):
        for k in range(CHUNK):
            base = k * HIST
            dsls = [pl.ds(d * LANES, LANES) for d in range(DIM // LANES)]
            accs = [buf[base, dsl] for dsl in dsls]
            for r in range(1, HIST):
                for d, dsl in enumerate(dsls):
                    accs[d] = accs[d] + buf[base + r, dsl]
            for d, dsl in enumerate(dsls):
                out_v[j * CHUNK + k, dsl] = accs[d] * INV_HIST

    start_gather(0, rows0, sem0)

    @pl.loop(0, NCHUNK, step=2)
    def _(j):
        start_gather(j + 1, rows1, sem1)
        wait_gather(j, rows0, sem0)
        reduce_chunk(j, rows0)

        @pl.when(j + 2 < NCHUNK)
        def _():
            start_gather(j + 2, rows0, sem0)

        wait_gather(j + 1, rows1, sem1)
        reduce_chunk(j + 1, rows1)

    base = wid * SAMPLES_PER_TILE
    pltpu.sync_copy(out_v, out_hbm.at[pl.ds(base, SAMPLES_PER_TILE)])


@jax.jit
def kernel(xs, table):
    xs = jnp.reshape(xs.astype(jnp.int32), (NUM_WORKERS, NCHUNK, IDX_PER_CHUNK))
    mesh = plsc.VectorSubcoreMesh(core_axis_name="c", subcore_axis_name="s")
    run = pl.kernel(
        _sc_body,
        out_type=jax.ShapeDtypeStruct((BATCH, DIM), jnp.float32),
        mesh=mesh,
        compiler_params=pltpu.CompilerParams(use_tc_tiling_on_sc=False),
        scratch_types=[
            pltpu.VMEM((NCHUNK, IDX_PER_CHUNK), jnp.int32),
            pltpu.VMEM((IDX_PER_CHUNK, DIM), jnp.float32),
            pltpu.VMEM((IDX_PER_CHUNK, DIM), jnp.float32),
            pltpu.VMEM((SAMPLES_PER_TILE, DIM), jnp.float32),
            pltpu.SemaphoreType.DMA,
            pltpu.SemaphoreType.DMA,
        ],
    )
    return run(xs, table)
